# baseline (speedup 1.0000x reference)
import functools

import jax
import jax.numpy as jnp
from jax import lax
from jax.experimental import pallas as pl
from jax.experimental.pallas import tpu as pltpu
from jax.experimental.pallas import tpu_sc as plsc

N_NODES = 100000
DIM = 16
NC = 2
NS = 16
NW = NC * NS
CHUNK = 512
N_ACC = 102400
ROWS_PER_TILE = N_ACC // NS
DUMMY_ROW = N_NODES


def _sc_scatter_pass(table, src_r, dst_r, n_super):
    mesh = plsc.VectorSubcoreMesh(core_axis_name="c", subcore_axis_name="s")

    @functools.partial(
        pl.kernel,
        out_type=jax.ShapeDtypeStruct((NC, N_ACC, DIM), jnp.float32),
        mesh=mesh,
        scratch_types=[
            pltpu.VMEM_SHARED((N_ACC, DIM), jnp.float32),
            pltpu.VMEM((2, CHUNK), jnp.int32),
            pltpu.VMEM((2, CHUNK), jnp.int32),
            pltpu.VMEM((2, CHUNK, DIM), jnp.float32),
            pltpu.SemaphoreType.DMA((2,)),
            pltpu.SemaphoreType.DMA((2,)),
        ],
        compiler_params=pltpu.CompilerParams(use_tc_tiling_on_sc=False),
    )
    def body(table_hbm, src_hbm, dst_hbm, out_hbm, acc, sbuf, dbuf, rows, gsem, ssem):
        cid = lax.axis_index("c")
        sid = lax.axis_index("s")
        wid = sid * NC + cid

        for i in range(128):
            rows[0, i, :] = jnp.zeros((DIM,), jnp.float32)
        base = sid * ROWS_PER_TILE
        for j in range(ROWS_PER_TILE // 128):
            pltpu.sync_copy(
                rows.at[0, pl.ds(0, 128)], acc.at[pl.ds(base + j * 128, 128)]
            )
        plsc.subcore_barrier()

        def stage_and_fire(sc, slot):
            pltpu.sync_copy(src_hbm.at[wid, sc], sbuf.at[slot])
            pltpu.sync_copy(dst_hbm.at[wid, sc], dbuf.at[slot])
            pltpu.async_copy(table_hbm.at[sbuf.at[slot]], rows.at[slot], gsem.at[slot])

        stage_and_fire(0, 0)

        def superchunk(sc, carry):
            p = lax.rem(sc, 2)
            q = 1 - p

            @pl.when(sc + 1 < n_super)
            def _():
                stage_and_fire(sc + 1, q)

            pltpu.make_async_copy(
                table_hbm.at[sbuf.at[p]], rows.at[p], gsem.at[p]
            ).wait()
            pltpu.async_copy(
                rows.at[p], acc.at[dbuf.at[p]], ssem.at[p], add=True
            ).wait()
            return carry

        lax.fori_loop(0, n_super, superchunk, 0)
        plsc.subcore_barrier()
        pltpu.sync_copy(
            acc.at[pl.ds(base, ROWS_PER_TILE)],
            out_hbm.at[cid, pl.ds(base, ROWS_PER_TILE)],
        )

    return body(table, src_r, dst_r)


def _tc_affine(partials, w, b, relu):
    blk = 4096

    def body(p_ref, w_ref, b_ref, o_ref):
        p = p_ref[...]
        z = jnp.dot(p[0] + p[1], w_ref[...], preferred_element_type=jnp.float32)
        z = z + b_ref[...]
        o_ref[...] = jnp.maximum(z, 0.0) if relu else z

    return pl.pallas_call(
        body,
        grid=(N_ACC // blk,),
        in_specs=[
            pl.BlockSpec((NC, blk, DIM), lambda i: (0, i, 0)),
            pl.BlockSpec((DIM, DIM), lambda i: (0, 0)),
            pl.BlockSpec((1, DIM), lambda i: (0, 0)),
        ],
        out_specs=pl.BlockSpec((blk, DIM), lambda i: (i, 0)),
        out_shape=jax.ShapeDtypeStruct((N_ACC, DIM), jnp.float32),
    )(partials, w, b.reshape(1, DIM))


def kernel(entity_emb, W1, b1, W2, b2, edge_index):
    n_edges = edge_index.shape[1]
    per_w = NW * CHUNK
    n_super = ((n_edges + per_w - 1) // per_w + 7) // 8 * 8
    e_pad = n_super * per_w
    pad = e_pad - n_edges

    src = edge_index[0]
    dst = edge_index[1]
    if pad:
        pad_dst = DUMMY_ROW + jnp.arange(pad, dtype=jnp.int32) % (N_ACC - N_NODES)
        src = jnp.concatenate([src, jnp.zeros((pad,), jnp.int32)])
        dst = jnp.concatenate([dst, pad_dst])
    src_r = src.reshape(NW, n_super, CHUNK)
    dst_r = dst.reshape(NW, n_super, CHUNK)

    p1 = _sc_scatter_pass(entity_emb, src_r, dst_r, n_super)
    x = _tc_affine(p1, W1, b1, relu=True)
    p2 = _sc_scatter_pass(x, src_r, dst_r, n_super)
    out = _tc_affine(p2, W2, b2, relu=False)
    return out[:N_NODES]


# speedup vs baseline: 1.6306x; 1.6306x over previous
import functools

import jax
import jax.numpy as jnp
from jax import lax
from jax.experimental import pallas as pl
from jax.experimental.pallas import tpu as pltpu
from jax.experimental.pallas import tpu_sc as plsc

N_NODES = 100000
DIM = 16
NC = 2
NS = 16
NW = NC * NS
CHUNK = 512
N_ACC = 102400
ROWS_PER_TILE = N_ACC // NS
DUMMY_ROW = N_NODES


def _sc_scatter_pass(table, src_r, dst_r, n_super):
    mesh = plsc.VectorSubcoreMesh(core_axis_name="c", subcore_axis_name="s")

    @functools.partial(
        pl.kernel,
        out_type=jax.ShapeDtypeStruct((NC, N_ACC, DIM), jnp.float32),
        mesh=mesh,
        scratch_types=[
            pltpu.VMEM_SHARED((N_ACC, DIM), jnp.float32),
            pltpu.VMEM((2, CHUNK), jnp.int32),
            pltpu.VMEM((2, CHUNK), jnp.int32),
            pltpu.VMEM((2, CHUNK, DIM), jnp.float32),
            pltpu.SemaphoreType.DMA((2,)),
            pltpu.SemaphoreType.DMA((2,)),
        ],
        compiler_params=pltpu.CompilerParams(use_tc_tiling_on_sc=False),
    )
    def body(table_hbm, src_hbm, dst_hbm, out_hbm, acc, sbuf, dbuf, rows, gsem, ssem):
        cid = lax.axis_index("c")
        sid = lax.axis_index("s")
        wid = sid * NC + cid

        for i in range(128):
            rows[0, i, :] = jnp.zeros((DIM,), jnp.float32)
        base = sid * ROWS_PER_TILE
        for j in range(ROWS_PER_TILE // 128):
            pltpu.sync_copy(
                rows.at[0, pl.ds(0, 128)], acc.at[pl.ds(base + j * 128, 128)]
            )
        plsc.subcore_barrier()

        def stage_and_fire(sc, slot):
            pltpu.sync_copy(src_hbm.at[wid, sc], sbuf.at[slot])
            pltpu.sync_copy(dst_hbm.at[wid, sc], dbuf.at[slot])
            pltpu.async_copy(table_hbm.at[sbuf.at[slot]], rows.at[slot], gsem.at[slot])

        stage_and_fire(0, 0)

        def superchunk(sc, carry):
            p = lax.rem(sc, 2)
            q = 1 - p

            @pl.when(sc + 1 < n_super)
            def _():
                stage_and_fire(sc + 1, q)

            pltpu.make_async_copy(
                table_hbm.at[sbuf.at[p]], rows.at[p], gsem.at[p]
            ).wait()
            pltpu.async_copy(
                rows.at[p], acc.at[dbuf.at[p]], ssem.at[p], add=True
            ).wait()
            return carry

        lax.fori_loop(0, n_super, superchunk, 0)
        plsc.subcore_barrier()
        pltpu.sync_copy(
            acc.at[pl.ds(base, ROWS_PER_TILE)],
            out_hbm.at[cid, pl.ds(base, ROWS_PER_TILE)],
        )

    return body(table, src_r, dst_r)


def _tc_affine(partials, w, b, relu):
    blk = 4096

    def body(p_ref, w_ref, b_ref, o_ref):
        p = p_ref[...]
        z = jnp.dot(p[0] + p[1], w_ref[...], preferred_element_type=jnp.float32)
        z = z + b_ref[...]
        o_ref[...] = jnp.maximum(z, 0.0) if relu else z

    return pl.pallas_call(
        body,
        grid=(N_ACC // blk,),
        in_specs=[
            pl.BlockSpec((NC, blk, DIM), lambda i: (0, i, 0)),
            pl.BlockSpec((DIM, DIM), lambda i: (0, 0)),
            pl.BlockSpec((1, DIM), lambda i: (0, 0)),
        ],
        out_specs=pl.BlockSpec((blk, DIM), lambda i: (i, 0)),
        out_shape=jax.ShapeDtypeStruct((N_ACC, DIM), jnp.float32),
    )(partials, w, b.reshape(1, DIM))


def kernel(entity_emb, W1, b1, W2, b2, edge_index):
    n_edges = edge_index.shape[1]
    per_w = NW * CHUNK
    n_super = ((n_edges + per_w - 1) // per_w + 7) // 8 * 8
    e_pad = n_super * per_w
    pad = e_pad - n_edges

    src = edge_index[0]
    dst = edge_index[1]
    if pad:
        pad_idx = jnp.arange(pad, dtype=jnp.int32)
        pad_dst = DUMMY_ROW + pad_idx % (N_ACC - N_NODES)
        pad_src = (pad_idx * 127) % N_NODES
        src = jnp.concatenate([src, pad_src])
        dst = jnp.concatenate([dst, pad_dst])
    src_r = src.reshape(NW, n_super, CHUNK)
    dst_r = dst.reshape(NW, n_super, CHUNK)

    p1 = _sc_scatter_pass(entity_emb, src_r, dst_r, n_super)
    x = _tc_affine(p1, W1, b1, relu=True)
    p2 = _sc_scatter_pass(x, src_r, dst_r, n_super)
    out = _tc_affine(p2, W2, b2, relu=False)
    return out[:N_NODES]


# speedup vs baseline: 1.6449x; 1.0088x over previous
import functools

import jax
import jax.numpy as jnp
from jax import lax
from jax.experimental import pallas as pl
from jax.experimental.pallas import tpu as pltpu
from jax.experimental.pallas import tpu_sc as plsc

N_NODES = 100000
DIM = 16
NC = 2
NS = 16
NW = NC * NS
CHUNK = 512
N_ACC = 100352
ROWS_PER_TILE = N_ACC // NS
NSLOT = 3
DUMMY_ROW = N_NODES


def _sc_scatter_pass(table, src_r, dst_r, n_super):
    mesh = plsc.VectorSubcoreMesh(core_axis_name="c", subcore_axis_name="s")

    @functools.partial(
        pl.kernel,
        out_type=jax.ShapeDtypeStruct((NC, N_ACC, DIM), jnp.float32),
        mesh=mesh,
        scratch_types=[
            pltpu.VMEM_SHARED((N_ACC, DIM), jnp.float32),
            pltpu.VMEM((NSLOT, CHUNK), jnp.int32),
            pltpu.VMEM((NSLOT, CHUNK), jnp.int32),
            pltpu.VMEM((NSLOT, CHUNK, DIM), jnp.float32),
            pltpu.SemaphoreType.DMA((NSLOT,)),
            pltpu.SemaphoreType.DMA((NSLOT,)),
        ],
        compiler_params=pltpu.CompilerParams(use_tc_tiling_on_sc=False),
    )
    def body(table_hbm, src_hbm, dst_hbm, out_hbm, acc, sbuf, dbuf, rows, gsem, ssem):
        cid = lax.axis_index("c")
        sid = lax.axis_index("s")
        wid = sid * NC + cid

        for i in range(128):
            rows[0, i, :] = jnp.zeros((DIM,), jnp.float32)
        base = sid * ROWS_PER_TILE
        for j in range(ROWS_PER_TILE // 128):
            pltpu.sync_copy(
                rows.at[0, pl.ds(0, 128)], acc.at[pl.ds(base + j * 128, 128)]
            )
        plsc.subcore_barrier()

        def stage_and_fire(sc, slot):
            pltpu.sync_copy(src_hbm.at[wid, sc], sbuf.at[slot])
            pltpu.sync_copy(dst_hbm.at[wid, sc], dbuf.at[slot])
            pltpu.async_copy(table_hbm.at[sbuf.at[slot]], rows.at[slot], gsem.at[slot])

        stage_and_fire(0, 0)
        stage_and_fire(1, 1)

        def superchunk(sc, carry):
            p = lax.rem(sc, NSLOT)

            @pl.when(sc + 2 < n_super)
            def _():
                r = lax.rem(sc + 2, NSLOT)

                @pl.when(sc >= 1)
                def _():
                    pltpu.make_async_copy(
                        rows.at[r], acc.at[dbuf.at[r]], ssem.at[r]
                    ).wait()

                stage_and_fire(sc + 2, r)

            pltpu.make_async_copy(
                table_hbm.at[sbuf.at[p]], rows.at[p], gsem.at[p]
            ).wait()
            pltpu.async_copy(rows.at[p], acc.at[dbuf.at[p]], ssem.at[p], add=True)
            return carry

        lax.fori_loop(0, n_super, superchunk, 0)
        for j in range(min(NSLOT, 3)):
            pltpu.make_async_copy(
                rows.at[j], acc.at[dbuf.at[j]], ssem.at[j]
            ).wait()
        plsc.subcore_barrier()
        pltpu.sync_copy(
            acc.at[pl.ds(base, ROWS_PER_TILE)],
            out_hbm.at[cid, pl.ds(base, ROWS_PER_TILE)],
        )

    return body(table, src_r, dst_r)


def _tc_affine(partials, w, b, relu):
    blk = 4096

    def body(p_ref, w_ref, b_ref, o_ref):
        p = p_ref[...]
        z = jnp.dot(p[0] + p[1], w_ref[...], preferred_element_type=jnp.float32)
        z = z + b_ref[...]
        o_ref[...] = jnp.maximum(z, 0.0) if relu else z

    return pl.pallas_call(
        body,
        grid=(N_ACC // blk,),
        in_specs=[
            pl.BlockSpec((NC, blk, DIM), lambda i: (0, i, 0)),
            pl.BlockSpec((DIM, DIM), lambda i: (0, 0)),
            pl.BlockSpec((1, DIM), lambda i: (0, 0)),
        ],
        out_specs=pl.BlockSpec((blk, DIM), lambda i: (i, 0)),
        out_shape=jax.ShapeDtypeStruct((N_ACC, DIM), jnp.float32),
    )(partials, w, b.reshape(1, DIM))


def kernel(entity_emb, W1, b1, W2, b2, edge_index):
    n_edges = edge_index.shape[1]
    per_w = NW * CHUNK
    n_super = ((n_edges + per_w - 1) // per_w + 7) // 8 * 8
    e_pad = n_super * per_w
    pad = e_pad - n_edges

    src = edge_index[0]
    dst = edge_index[1]
    if pad:
        pad_idx = jnp.arange(pad, dtype=jnp.int32)
        pad_dst = DUMMY_ROW + pad_idx % (N_ACC - N_NODES)
        pad_src = (pad_idx * 127) % N_NODES
        src = jnp.concatenate([src, pad_src])
        dst = jnp.concatenate([dst, pad_dst])
    src_r = src.reshape(NW, n_super, CHUNK)
    dst_r = dst.reshape(NW, n_super, CHUNK)

    p1 = _sc_scatter_pass(entity_emb, src_r, dst_r, n_super)
    x = _tc_affine(p1, W1, b1, relu=True)
    p2 = _sc_scatter_pass(x, src_r, dst_r, n_super)
    out = _tc_affine(p2, W2, b2, relu=False)
    return out[:N_NODES]
